# baseline (speedup 1.0000x reference)
import functools

import jax
import jax.numpy as jnp
from jax import lax
from jax.experimental import pallas as pl
from jax.experimental.pallas import tpu as pltpu
from jax.experimental.pallas import tpu_sc as plsc

B, L, F = 1024, 50, 64
FW = F // 2
N_BAG = 26
VOCAB = 1000000
ROWS = B * L

NC, NS = 2, 16
NW = NC * NS
ROWS_PER_W = ROWS // NW
GRP = 4
IDX_PER_GRP = GRP * N_BAG
GRPS_PER_W = ROWS_PER_W // GRP
IDX_ROWS = ROWS * N_BAG // IDX_PER_GRP

NBUF = 4

CB = 512
NFULL = VOCAB // CB
TAIL0 = NFULL * CB
TAILN = VOCAB - TAIL0
NSLOT = 62


def _prep_body(tt_hbm, lin_hbm, in_v0, in_v1, out_v0, out_v1, in_t, out_t, isem, osem, tsem):
    in_v = (in_v0, in_v1)
    out_v = (out_v0, out_v1)
    wid = lax.axis_index("s") * NC + lax.axis_index("c")
    iota = lax.iota(jnp.int32, 16)

    def in_copy(i, b):
        ch = wid + NW * i
        return pltpu.make_async_copy(
            tt_hbm.at[:, pl.ds(ch * CB, CB)], in_v[b], isem.at[b])

    def out_copy(i, b):
        ch = wid + NW * i
        return pltpu.make_async_copy(
            out_v[b], lin_hbm.at[pl.ds(ch * CB * FW, CB * FW)], osem.at[b])

    def valid(i):
        return wid + NW * i < NFULL

    def convert(src, dst, r):
        col = jnp.full((16,), r, jnp.int32)
        for k2 in range(2):
            lo = plsc.load_gather(src, [k2 * 32 + iota, col])
            hi = plsc.load_gather(src, [k2 * 32 + 16 + iota, col])
            packed = plsc.pack(lo, hi, format=plsc.PackFormat.INTERLEAVED,
                               preferred_element_type=jnp.bfloat16)
            dst[pl.ds(r * FW + k2 * 16, 16)] = plsc.bitcast(packed, jnp.float32)

    for b in range(2):
        @pl.when(valid(b))
        def _():
            in_copy(b, b).start()

    def pair(p, _):
        for b in range(2):
            i = p * 2 + b
            @pl.when(valid(i))
            def _():
                in_copy(i, b).wait()

                @pl.when(i >= 2)
                def _():
                    out_copy(i - 2, b).wait()

                def rows8(r8, _):
                    for rr in range(8):
                        convert(in_v[b], out_v[b], r8 * 8 + rr)
                    return 0
                lax.fori_loop(0, CB // 8, rows8, 0)

                out_copy(i, b).start()

                @pl.when(valid(i + 2))
                def _():
                    in_copy(i + 2, b).start()
        return 0

    lax.fori_loop(0, NSLOT // 2, pair, 0)

    @pl.when(valid(NSLOT - 2))
    def _():
        out_copy(NSLOT - 2, 0).wait()

    @pl.when(valid(NSLOT - 1))
    def _():
        out_copy(NSLOT - 1, 1).wait()

    @pl.when(jnp.logical_and(jnp.logical_not(valid(NSLOT - 1)), valid(NSLOT - 3)))
    def _():
        out_copy(NSLOT - 3, 1).wait()

    @pl.when(wid == 0)
    def _():
        pltpu.make_async_copy(
            tt_hbm.at[:, pl.ds(TAIL0, TAILN)], in_t, tsem).start()
        pltpu.make_async_copy(
            tt_hbm.at[:, pl.ds(TAIL0, TAILN)], in_t, tsem).wait()

        def trow(r, _):
            convert(in_t, out_t, r)
            return 0
        lax.fori_loop(0, TAILN, trow, 0)
        pltpu.make_async_copy(
            out_t, lin_hbm.at[pl.ds(TAIL0 * FW, TAILN * FW)], tsem).start()
        pltpu.make_async_copy(
            out_t, lin_hbm.at[pl.ds(TAIL0 * FW, TAILN * FW)], tsem).wait()


def _bag_body(idx_hbm, table_hbm, out_hbm, idx_v, rows_v, out_v, gsem, osem):
    wid = lax.axis_index("s") * NC + lax.axis_index("c")
    base_row = wid * ROWS_PER_W
    pltpu.sync_copy(idx_hbm.at[pl.ds(wid * GRPS_PER_W, GRPS_PER_W)], idx_v)

    def gather(u, b):
        return pltpu.make_async_copy(
            table_hbm.at[idx_v.at[u]], rows_v.at[b], gsem.at[b])

    def out_write(u, b):
        return pltpu.make_async_copy(
            out_v.at[b], out_hbm.at[pl.ds(base_row + u * GRP, GRP)], osem.at[b])

    for b in range(NBUF):
        gather(b, b).start()

    def outer(t, _):
        u0 = t * NBUF
        for b in range(NBUF):
            u = u0 + b
            gather(u, b).wait()

            @pl.when(t > 0)
            def _():
                out_write(u - NBUF, b).wait()

            for r in range(GRP):
                for k2 in range(2):
                    sl = pl.ds(k2 * 16, 16)
                    a0, b0 = plsc.unpack(
                        plsc.bitcast(rows_v[b, r * N_BAG, sl], jnp.bfloat16),
                        format=plsc.PackFormat.INTERLEAVED,
                        preferred_element_type=jnp.float32)
                    for j in range(1, N_BAG):
                        aj, bj = plsc.unpack(
                            plsc.bitcast(rows_v[b, r * N_BAG + j, sl], jnp.bfloat16),
                            format=plsc.PackFormat.INTERLEAVED,
                            preferred_element_type=jnp.float32)
                        a0 = a0 + aj
                        b0 = b0 + bj
                    out_v[b, r, pl.ds(k2 * 32, 16)] = a0
                    out_v[b, r, pl.ds(k2 * 32 + 16, 16)] = b0

            @pl.when(u + NBUF < GRPS_PER_W)
            def _():
                gather(u + NBUF, b).start()

            out_write(u, b).start()
        return 0

    lax.fori_loop(0, GRPS_PER_W // NBUF, outer, 0)

    for b in range(NBUF):
        out_write(GRPS_PER_W - NBUF + b, b).wait()


@functools.partial(jax.jit, static_argnums=())
def _run(idx, table_t):
    mesh = plsc.VectorSubcoreMesh(core_axis_name="c", subcore_axis_name="s")
    prep = functools.partial(
        pl.kernel,
        mesh=mesh,
        out_type=jax.ShapeDtypeStruct((VOCAB * FW,), jnp.float32),
        scratch_types=[
            pltpu.VMEM((F, CB), jnp.float32),
            pltpu.VMEM((F, CB), jnp.float32),
            pltpu.VMEM((CB * FW,), jnp.float32),
            pltpu.VMEM((CB * FW,), jnp.float32),
            pltpu.VMEM((F, TAILN), jnp.float32),
            pltpu.VMEM((TAILN * FW,), jnp.float32),
            pltpu.SemaphoreType.DMA((2,)),
            pltpu.SemaphoreType.DMA((2,)),
            pltpu.SemaphoreType.DMA,
        ],
        compiler_params=pltpu.CompilerParams(
            use_tc_tiling_on_sc=True, needs_layout_passes=False),
    )(_prep_body)
    table_lin = prep(table_t).reshape(VOCAB, FW)

    bag = functools.partial(
        pl.kernel,
        mesh=mesh,
        out_type=jax.ShapeDtypeStruct((ROWS, F), jnp.float32),
        scratch_types=[
            pltpu.VMEM((GRPS_PER_W, IDX_PER_GRP), jnp.int32),
            pltpu.VMEM((NBUF, IDX_PER_GRP, FW), jnp.float32),
            pltpu.VMEM((NBUF, GRP, F), jnp.float32),
            pltpu.SemaphoreType.DMA((NBUF,)),
            pltpu.SemaphoreType.DMA((NBUF,)),
        ],
        compiler_params=pltpu.CompilerParams(
            use_tc_tiling_on_sc=False, needs_layout_passes=False),
    )(_bag_body)
    return bag(idx, table_lin)


def kernel(seqs, context_cat_inputs, table):
    b, l, f = seqs.shape
    idx = context_cat_inputs.astype(jnp.int32).reshape(IDX_ROWS, IDX_PER_GRP)
    out = _run(idx, table.T)
    return out.reshape(b, l, f)


# speedup vs baseline: 1.9212x; 1.9212x over previous
import functools

import jax
import jax.numpy as jnp
from jax import lax
from jax.experimental import pallas as pl
from jax.experimental.pallas import tpu as pltpu
from jax.experimental.pallas import tpu_sc as plsc

B, L, F = 1024, 50, 64
FW = F // 2
N_BAG = 26
VOCAB = 1000000
ROWS = B * L

NC, NS = 2, 16
NW = NC * NS
ROWS_PER_W = ROWS // NW
GRP = 4
IDX_PER_GRP = GRP * N_BAG
GRPS_PER_W = ROWS_PER_W // GRP
IDX_ROWS = ROWS * N_BAG // IDX_PER_GRP

NBUF = 4

CB = 256
NFULL = VOCAB // CB
TAIL0 = NFULL * CB
TAILN = VOCAB - TAIL0
NSLOT = 124


def _prep_body(tab_hbm, lin_hbm, in_v0, in_v1, out_v0, out_v1, in_t, out_t, isem, osem, tsem):
    in_v = (in_v0, in_v1)
    out_v = (out_v0, out_v1)
    wid = lax.axis_index("s") * NC + lax.axis_index("c")

    def in_copy(i, b):
        ch = wid + NW * i
        return pltpu.make_async_copy(
            tab_hbm.at[pl.ds(ch * CB, CB)], in_v[b], isem.at[b])

    def out_copy(i, b):
        ch = wid + NW * i
        return pltpu.make_async_copy(
            out_v[b], lin_hbm.at[pl.ds(ch * CB * FW, CB * FW)], osem.at[b])

    def valid(i):
        return wid + NW * i < NFULL

    def convert(src_ref, dst_ref, r):
        for k2 in range(2):
            x = src_ref[r, pl.ds(k2 * 32, 16)]
            y = src_ref[r, pl.ds(k2 * 32 + 16, 16)]
            packed = plsc.pack(x, y, format=plsc.PackFormat.INTERLEAVED,
                               preferred_element_type=jnp.bfloat16)
            dst_ref[pl.ds(r * FW + k2 * 16, 16)] = plsc.bitcast(packed, jnp.float32)

    for b in range(2):
        @pl.when(valid(b))
        def _():
            in_copy(b, b).start()

    def pair(p, _):
        for b in range(2):
            i = p * 2 + b
            @pl.when(valid(i))
            def _():
                in_copy(i, b).wait()

                @pl.when(i >= 2)
                def _():
                    out_copy(i - 2, b).wait()

                def rows8(r8, _):
                    for rr in range(8):
                        convert(in_v[b], out_v[b], r8 * 8 + rr)
                    return 0
                lax.fori_loop(0, CB // 8, rows8, 0)

                out_copy(i, b).start()

                @pl.when(valid(i + 2))
                def _():
                    in_copy(i + 2, b).start()
        return 0

    lax.fori_loop(0, NSLOT // 2, pair, 0)

    @pl.when(valid(NSLOT - 2))
    def _():
        out_copy(NSLOT - 2, 0).wait()

    @pl.when(valid(NSLOT - 1))
    def _():
        out_copy(NSLOT - 1, 1).wait()

    @pl.when(jnp.logical_and(jnp.logical_not(valid(NSLOT - 1)), valid(NSLOT - 3)))
    def _():
        out_copy(NSLOT - 3, 1).wait()

    @pl.when(wid == 0)
    def _():
        pltpu.make_async_copy(
            tab_hbm.at[pl.ds(TAIL0, TAILN)], in_t, tsem).start()
        pltpu.make_async_copy(
            tab_hbm.at[pl.ds(TAIL0, TAILN)], in_t, tsem).wait()

        def trow(r, _):
            convert(in_t, out_t, r)
            return 0
        lax.fori_loop(0, TAILN, trow, 0)
        pltpu.make_async_copy(
            out_t, lin_hbm.at[pl.ds(TAIL0 * FW, TAILN * FW)], tsem).start()
        pltpu.make_async_copy(
            out_t, lin_hbm.at[pl.ds(TAIL0 * FW, TAILN * FW)], tsem).wait()


def _bag_body(idx_hbm, table_hbm, out_hbm, idx_v, rows_v, out_v, gsem, osem):
    wid = lax.axis_index("s") * NC + lax.axis_index("c")
    base_row = wid * ROWS_PER_W
    pltpu.sync_copy(idx_hbm.at[pl.ds(wid * GRPS_PER_W, GRPS_PER_W)], idx_v)

    def gather(u, b):
        return pltpu.make_async_copy(
            table_hbm.at[idx_v.at[u]], rows_v.at[b], gsem.at[b])

    def out_write(u, b):
        return pltpu.make_async_copy(
            out_v.at[b], out_hbm.at[pl.ds(base_row + u * GRP, GRP)], osem.at[b])

    for b in range(NBUF):
        gather(b, b).start()

    def outer(t, _):
        u0 = t * NBUF
        for b in range(NBUF):
            u = u0 + b
            gather(u, b).wait()

            @pl.when(t > 0)
            def _():
                out_write(u - NBUF, b).wait()

            for r in range(GRP):
                for k2 in range(2):
                    sl = pl.ds(k2 * 16, 16)
                    a0, b0 = plsc.unpack(
                        plsc.bitcast(rows_v[b, r * N_BAG, sl], jnp.bfloat16),
                        format=plsc.PackFormat.INTERLEAVED,
                        preferred_element_type=jnp.float32)
                    for j in range(1, N_BAG):
                        aj, bj = plsc.unpack(
                            plsc.bitcast(rows_v[b, r * N_BAG + j, sl], jnp.bfloat16),
                            format=plsc.PackFormat.INTERLEAVED,
                            preferred_element_type=jnp.float32)
                        a0 = a0 + aj
                        b0 = b0 + bj
                    out_v[b, r, pl.ds(k2 * 32, 16)] = a0
                    out_v[b, r, pl.ds(k2 * 32 + 16, 16)] = b0

            @pl.when(u + NBUF < GRPS_PER_W)
            def _():
                gather(u + NBUF, b).start()

            out_write(u, b).start()
        return 0

    lax.fori_loop(0, GRPS_PER_W // NBUF, outer, 0)

    for b in range(NBUF):
        out_write(GRPS_PER_W - NBUF + b, b).wait()


@functools.partial(jax.jit, static_argnums=())
def _run(idx, table_t):
    mesh = plsc.VectorSubcoreMesh(core_axis_name="c", subcore_axis_name="s")
    prep = functools.partial(
        pl.kernel,
        mesh=mesh,
        out_type=jax.ShapeDtypeStruct((VOCAB * FW,), jnp.float32),
        scratch_types=[
            pltpu.VMEM((CB, F), jnp.float32),
            pltpu.VMEM((CB, F), jnp.float32),
            pltpu.VMEM((CB * FW,), jnp.float32),
            pltpu.VMEM((CB * FW,), jnp.float32),
            pltpu.VMEM((TAILN, F), jnp.float32),
            pltpu.VMEM((TAILN * FW,), jnp.float32),
            pltpu.SemaphoreType.DMA((2,)),
            pltpu.SemaphoreType.DMA((2,)),
            pltpu.SemaphoreType.DMA,
        ],
        compiler_params=pltpu.CompilerParams(
            use_tc_tiling_on_sc=True, needs_layout_passes=False),
    )(_prep_body)
    table_lin = prep(table_t).reshape(VOCAB, FW)

    bag = functools.partial(
        pl.kernel,
        mesh=mesh,
        out_type=jax.ShapeDtypeStruct((ROWS, F), jnp.float32),
        scratch_types=[
            pltpu.VMEM((GRPS_PER_W, IDX_PER_GRP), jnp.int32),
            pltpu.VMEM((NBUF, IDX_PER_GRP, FW), jnp.float32),
            pltpu.VMEM((NBUF, GRP, F), jnp.float32),
            pltpu.SemaphoreType.DMA((NBUF,)),
            pltpu.SemaphoreType.DMA((NBUF,)),
        ],
        compiler_params=pltpu.CompilerParams(
            use_tc_tiling_on_sc=False, needs_layout_passes=False),
    )(_bag_body)
    return bag(idx, table_lin)


def kernel(seqs, context_cat_inputs, table):
    b, l, f = seqs.shape
    idx = context_cat_inputs.astype(jnp.int32).reshape(IDX_ROWS, IDX_PER_GRP)
    out = _run(idx, table)
    return out.reshape(b, l, f)


# speedup vs baseline: 1.9251x; 1.0020x over previous
import functools

import jax
import jax.numpy as jnp
from jax import lax
from jax.experimental import pallas as pl
from jax.experimental.pallas import tpu as pltpu
from jax.experimental.pallas import tpu_sc as plsc

B, L, F = 1024, 50, 64
FW = F // 2
N_BAG = 26
VOCAB = 1000000
ROWS = B * L

NC, NS = 2, 16
NW = NC * NS
ROWS_PER_W = ROWS // NW
GRP = 4
IDX_PER_GRP = GRP * N_BAG
GRPS_PER_W = ROWS_PER_W // GRP
IDX_ROWS = ROWS * N_BAG // IDX_PER_GRP

NBUF = 4

CB = 256
NFULL = VOCAB // CB
TAIL0 = NFULL * CB
TAILN = VOCAB - TAIL0
NSLOT = 124


def _prep_body(tab_hbm, lin_hbm, in_v0, in_v1, out_v0, out_v1, in_t, out_t, isem, osem, tsem):
    in_v = (in_v0, in_v1)
    out_v = (out_v0, out_v1)
    wid = lax.axis_index("s") * NC + lax.axis_index("c")

    def in_copy(i, b):
        ch = wid + NW * i
        return pltpu.make_async_copy(
            tab_hbm.at[pl.ds(ch * CB, CB)], in_v[b], isem.at[b])

    def out_copy(i, b):
        ch = wid + NW * i
        return pltpu.make_async_copy(
            out_v[b], lin_hbm.at[pl.ds(ch * CB * FW, CB * FW)], osem.at[b])

    def valid(i):
        return wid + NW * i < NFULL

    def convert(src_ref, dst_ref, r):
        for k2 in range(2):
            x = src_ref[r, pl.ds(k2 * 32, 16)]
            y = src_ref[r, pl.ds(k2 * 32 + 16, 16)]
            packed = plsc.pack(x, y, format=plsc.PackFormat.INTERLEAVED,
                               preferred_element_type=jnp.bfloat16)
            dst_ref[pl.ds(r * FW + k2 * 16, 16)] = plsc.bitcast(packed, jnp.float32)

    for b in range(2):
        @pl.when(valid(b))
        def _():
            in_copy(b, b).start()

    def pair(p, _):
        for b in range(2):
            i = p * 2 + b
            @pl.when(valid(i))
            def _():
                in_copy(i, b).wait()

                @pl.when(i >= 2)
                def _():
                    out_copy(i - 2, b).wait()

                def rows8(r8, _):
                    for rr in range(8):
                        convert(in_v[b], out_v[b], r8 * 8 + rr)
                    return 0
                lax.fori_loop(0, CB // 8, rows8, 0)

                out_copy(i, b).start()

                @pl.when(valid(i + 2))
                def _():
                    in_copy(i + 2, b).start()
        return 0

    lax.fori_loop(0, NSLOT // 2, pair, 0)

    for b in range(2):
        last = NSLOT - 2 + b
        @pl.when(valid(last))
        def _():
            out_copy(last, b).wait()

        @pl.when(jnp.logical_and(jnp.logical_not(valid(last)), valid(last - 2)))
        def _():
            out_copy(last - 2, b).wait()

        @pl.when(jnp.logical_and(jnp.logical_not(valid(last - 2)), valid(last - 4)))
        def _():
            out_copy(last - 4, b).wait()

    @pl.when(wid == 0)
    def _():
        pltpu.make_async_copy(
            tab_hbm.at[pl.ds(TAIL0, TAILN)], in_t, tsem).start()
        pltpu.make_async_copy(
            tab_hbm.at[pl.ds(TAIL0, TAILN)], in_t, tsem).wait()

        def trow(r, _):
            convert(in_t, out_t, r)
            return 0
        lax.fori_loop(0, TAILN, trow, 0)
        pltpu.make_async_copy(
            out_t, lin_hbm.at[pl.ds(TAIL0 * FW, TAILN * FW)], tsem).start()
        pltpu.make_async_copy(
            out_t, lin_hbm.at[pl.ds(TAIL0 * FW, TAILN * FW)], tsem).wait()


def _bag_body(idx_hbm, table_hbm, out_hbm, idx_v, rows_v, out_v, gsem, osem):
    wid = lax.axis_index("s") * NC + lax.axis_index("c")
    base_row = wid * ROWS_PER_W
    pltpu.sync_copy(idx_hbm.at[pl.ds(wid * GRPS_PER_W, GRPS_PER_W)], idx_v)

    def gather(u, b):
        return pltpu.make_async_copy(
            table_hbm.at[idx_v.at[u]], rows_v.at[b], gsem.at[b])

    def out_write(u, b):
        return pltpu.make_async_copy(
            out_v.at[b], out_hbm.at[pl.ds(base_row + u * GRP, GRP)], osem.at[b])

    for b in range(NBUF):
        gather(b, b).start()

    def outer(t, _):
        u0 = t * NBUF
        for b in range(NBUF):
            u = u0 + b
            gather(u, b).wait()

            @pl.when(t > 0)
            def _():
                out_write(u - NBUF, b).wait()

            for r in range(GRP):
                for k2 in range(2):
                    sl = pl.ds(k2 * 16, 16)
                    a0, b0 = plsc.unpack(
                        plsc.bitcast(rows_v[b, r * N_BAG, sl], jnp.bfloat16),
                        format=plsc.PackFormat.INTERLEAVED,
                        preferred_element_type=jnp.float32)
                    for j in range(1, N_BAG):
                        aj, bj = plsc.unpack(
                            plsc.bitcast(rows_v[b, r * N_BAG + j, sl], jnp.bfloat16),
                            format=plsc.PackFormat.INTERLEAVED,
                            preferred_element_type=jnp.float32)
                        a0 = a0 + aj
                        b0 = b0 + bj
                    out_v[b, r, pl.ds(k2 * 32, 16)] = a0
                    out_v[b, r, pl.ds(k2 * 32 + 16, 16)] = b0

            @pl.when(u + NBUF < GRPS_PER_W)
            def _():
                gather(u + NBUF, b).start()

            out_write(u, b).start()
        return 0

    lax.fori_loop(0, GRPS_PER_W // NBUF, outer, 0)

    for b in range(NBUF):
        out_write(GRPS_PER_W - NBUF + b, b).wait()


@functools.partial(jax.jit, static_argnums=())
def _run(idx, table_t):
    mesh = plsc.VectorSubcoreMesh(core_axis_name="c", subcore_axis_name="s")
    prep = functools.partial(
        pl.kernel,
        mesh=mesh,
        out_type=jax.ShapeDtypeStruct((VOCAB * FW,), jnp.float32),
        scratch_types=[
            pltpu.VMEM((CB, F), jnp.float32),
            pltpu.VMEM((CB, F), jnp.float32),
            pltpu.VMEM((CB * FW,), jnp.float32),
            pltpu.VMEM((CB * FW,), jnp.float32),
            pltpu.VMEM((TAILN, F), jnp.float32),
            pltpu.VMEM((TAILN * FW,), jnp.float32),
            pltpu.SemaphoreType.DMA((2,)),
            pltpu.SemaphoreType.DMA((2,)),
            pltpu.SemaphoreType.DMA,
        ],
        compiler_params=pltpu.CompilerParams(
            use_tc_tiling_on_sc=True, needs_layout_passes=False),
    )(_prep_body)
    table_lin = prep(table_t).reshape(VOCAB, FW)

    bag = functools.partial(
        pl.kernel,
        mesh=mesh,
        out_type=jax.ShapeDtypeStruct((ROWS, F), jnp.float32),
        scratch_types=[
            pltpu.VMEM((GRPS_PER_W, IDX_PER_GRP), jnp.int32),
            pltpu.VMEM((NBUF, IDX_PER_GRP, FW), jnp.float32),
            pltpu.VMEM((NBUF, GRP, F), jnp.float32),
            pltpu.SemaphoreType.DMA((NBUF,)),
            pltpu.SemaphoreType.DMA((NBUF,)),
        ],
        compiler_params=pltpu.CompilerParams(
            use_tc_tiling_on_sc=False, needs_layout_passes=False),
    )(_bag_body)
    return bag(idx, table_lin)


def kernel(seqs, context_cat_inputs, table):
    b, l, f = seqs.shape
    idx = context_cat_inputs.astype(jnp.int32).reshape(IDX_ROWS, IDX_PER_GRP)
    out = _run(idx, table)
    return out.reshape(b, l, f)


# speedup vs baseline: 1.9299x; 1.0025x over previous
import functools

import jax
import jax.numpy as jnp
from jax import lax
from jax.experimental import pallas as pl
from jax.experimental.pallas import tpu as pltpu
from jax.experimental.pallas import tpu_sc as plsc

B, L, F = 1024, 50, 64
FW = F // 2
N_BAG = 26
VOCAB = 1000000
ROWS = B * L

NC, NS = 2, 16
NW = NC * NS
ROWS_PER_W = ROWS // NW
GRP = 4
IDX_PER_GRP = GRP * N_BAG
GRPS_PER_W = ROWS_PER_W // GRP
IDX_ROWS = ROWS * N_BAG // IDX_PER_GRP

NBUF = 4

CB = 256
NFULL = VOCAB // CB
TAIL0 = NFULL * CB
TAILN = VOCAB - TAIL0
NSLOT = 124


def _prep_body(tab_hbm, lin_hbm, in_v0, in_v1, out_v0, out_v1, in_t, out_t, isem, osem, tsem):
    in_v = (in_v0, in_v1)
    out_v = (out_v0, out_v1)
    wid = lax.axis_index("s") * NC + lax.axis_index("c")

    def in_copy(i, b):
        ch = wid + NW * i
        return pltpu.make_async_copy(
            tab_hbm.at[pl.ds(ch * CB, CB)], in_v[b], isem.at[b])

    def out_copy(i, b):
        ch = wid + NW * i
        return pltpu.make_async_copy(
            out_v[b], lin_hbm.at[pl.ds(ch * CB * FW, CB * FW)], osem.at[b])

    def valid(i):
        return wid + NW * i < NFULL

    def convert(src_ref, dst_ref, r):
        for k2 in range(2):
            x = src_ref[r, pl.ds(k2 * 32, 16)]
            y = src_ref[r, pl.ds(k2 * 32 + 16, 16)]
            packed = plsc.pack(x, y, format=plsc.PackFormat.INTERLEAVED,
                               preferred_element_type=jnp.bfloat16)
            dst_ref[pl.ds(r * FW + k2 * 16, 16)] = plsc.bitcast(packed, jnp.float32)

    for b in range(2):
        @pl.when(valid(b))
        def _():
            in_copy(b, b).start()

    def pair(p, _):
        for b in range(2):
            i = p * 2 + b
            @pl.when(valid(i))
            def _():
                in_copy(i, b).wait()

                @pl.when(i >= 2)
                def _():
                    out_copy(i - 2, b).wait()

                def rows16(r16, _):
                    for rr in range(16):
                        convert(in_v[b], out_v[b], r16 * 16 + rr)
                    return 0
                lax.fori_loop(0, CB // 16, rows16, 0)

                out_copy(i, b).start()

                @pl.when(valid(i + 2))
                def _():
                    in_copy(i + 2, b).start()
        return 0

    lax.fori_loop(0, NSLOT // 2, pair, 0)

    for b in range(2):
        last = NSLOT - 2 + b
        @pl.when(valid(last))
        def _():
            out_copy(last, b).wait()

        @pl.when(jnp.logical_and(jnp.logical_not(valid(last)), valid(last - 2)))
        def _():
            out_copy(last - 2, b).wait()

        @pl.when(jnp.logical_and(jnp.logical_not(valid(last - 2)), valid(last - 4)))
        def _():
            out_copy(last - 4, b).wait()

    @pl.when(wid == 0)
    def _():
        pltpu.make_async_copy(
            tab_hbm.at[pl.ds(TAIL0, TAILN)], in_t, tsem).start()
        pltpu.make_async_copy(
            tab_hbm.at[pl.ds(TAIL0, TAILN)], in_t, tsem).wait()

        def trow(r, _):
            convert(in_t, out_t, r)
            return 0
        lax.fori_loop(0, TAILN, trow, 0)
        pltpu.make_async_copy(
            out_t, lin_hbm.at[pl.ds(TAIL0 * FW, TAILN * FW)], tsem).start()
        pltpu.make_async_copy(
            out_t, lin_hbm.at[pl.ds(TAIL0 * FW, TAILN * FW)], tsem).wait()


def _bag_body(idx_hbm, table_hbm, out_hbm, idx_v, rows_v, out_v, gsem, osem):
    wid = lax.axis_index("s") * NC + lax.axis_index("c")
    base_row = wid * ROWS_PER_W
    pltpu.sync_copy(idx_hbm.at[pl.ds(wid * GRPS_PER_W, GRPS_PER_W)], idx_v)

    def gather(u, b):
        return pltpu.make_async_copy(
            table_hbm.at[idx_v.at[u]], rows_v.at[b], gsem.at[b])

    def out_write(u, b):
        return pltpu.make_async_copy(
            out_v.at[b], out_hbm.at[pl.ds(base_row + u * GRP, GRP)], osem.at[b])

    for b in range(NBUF):
        gather(b, b).start()

    def outer(t, _):
        u0 = t * NBUF
        for b in range(NBUF):
            u = u0 + b
            gather(u, b).wait()

            @pl.when(t > 0)
            def _():
                out_write(u - NBUF, b).wait()

            for r in range(GRP):
                for k2 in range(2):
                    sl = pl.ds(k2 * 16, 16)
                    a0, b0 = plsc.unpack(
                        plsc.bitcast(rows_v[b, r * N_BAG, sl], jnp.bfloat16),
                        format=plsc.PackFormat.INTERLEAVED,
                        preferred_element_type=jnp.float32)
                    for j in range(1, N_BAG):
                        aj, bj = plsc.unpack(
                            plsc.bitcast(rows_v[b, r * N_BAG + j, sl], jnp.bfloat16),
                            format=plsc.PackFormat.INTERLEAVED,
                            preferred_element_type=jnp.float32)
                        a0 = a0 + aj
                        b0 = b0 + bj
                    out_v[b, r, pl.ds(k2 * 32, 16)] = a0
                    out_v[b, r, pl.ds(k2 * 32 + 16, 16)] = b0

            @pl.when(u + NBUF < GRPS_PER_W)
            def _():
                gather(u + NBUF, b).start()

            out_write(u, b).start()
        return 0

    lax.fori_loop(0, GRPS_PER_W // NBUF, outer, 0)

    for b in range(NBUF):
        out_write(GRPS_PER_W - NBUF + b, b).wait()


@functools.partial(jax.jit, static_argnums=())
def _run(idx, table_t):
    mesh = plsc.VectorSubcoreMesh(core_axis_name="c", subcore_axis_name="s")
    prep = functools.partial(
        pl.kernel,
        mesh=mesh,
        out_type=jax.ShapeDtypeStruct((VOCAB * FW,), jnp.float32),
        scratch_types=[
            pltpu.VMEM((CB, F), jnp.float32),
            pltpu.VMEM((CB, F), jnp.float32),
            pltpu.VMEM((CB * FW,), jnp.float32),
            pltpu.VMEM((CB * FW,), jnp.float32),
            pltpu.VMEM((TAILN, F), jnp.float32),
            pltpu.VMEM((TAILN * FW,), jnp.float32),
            pltpu.SemaphoreType.DMA((2,)),
            pltpu.SemaphoreType.DMA((2,)),
            pltpu.SemaphoreType.DMA,
        ],
        compiler_params=pltpu.CompilerParams(
            use_tc_tiling_on_sc=True, needs_layout_passes=False),
    )(_prep_body)
    table_lin = prep(table_t).reshape(VOCAB, FW)

    bag = functools.partial(
        pl.kernel,
        mesh=mesh,
        out_type=jax.ShapeDtypeStruct((ROWS, F), jnp.float32),
        scratch_types=[
            pltpu.VMEM((GRPS_PER_W, IDX_PER_GRP), jnp.int32),
            pltpu.VMEM((NBUF, IDX_PER_GRP, FW), jnp.float32),
            pltpu.VMEM((NBUF, GRP, F), jnp.float32),
            pltpu.SemaphoreType.DMA((NBUF,)),
            pltpu.SemaphoreType.DMA((NBUF,)),
        ],
        compiler_params=pltpu.CompilerParams(
            use_tc_tiling_on_sc=False, needs_layout_passes=False),
    )(_bag_body)
    return bag(idx, table_lin)


def kernel(seqs, context_cat_inputs, table):
    b, l, f = seqs.shape
    idx = context_cat_inputs.astype(jnp.int32).reshape(IDX_ROWS, IDX_PER_GRP)
    out = _run(idx, table)
    return out.reshape(b, l, f)
